# Initial kernel scaffold; baseline (speedup 1.0000x reference)
import functools

import jax
import jax.numpy as jnp
from jax import lax
from jax.experimental import pallas as pl
from jax.experimental.pallas import tpu as pltpu
from jax.experimental.pallas import tpu_sc as plsc

D = 128
_BN = 2000


def _rup(x, m):
    return (x + m - 1) // m * m


def _proj_student(x, W, b):
    n = x.shape[0]

    def body(x_ref, w_ref, b_ref, ys_ref, rs_ref, rd_ref):
        y = jnp.dot(x_ref[...], w_ref[...], preferred_element_type=jnp.float32)
        y = y + b_ref[...]
        ry = jnp.maximum(y, 0.0)
        rd_ref[...] = ry
        for j in range(8):
            ys_ref[j] = y[:, 16 * j:16 * (j + 1)]
            rs_ref[j] = ry[:, 16 * j:16 * (j + 1)]

    return pl.pallas_call(
        body,
        grid=(n // _BN,),
        in_specs=[
            pl.BlockSpec((_BN, D), lambda i: (i, 0)),
            pl.BlockSpec((D, D), lambda i: (0, 0)),
            pl.BlockSpec((1, D), lambda i: (0, 0)),
        ],
        out_specs=[
            pl.BlockSpec((8, _BN, 16), lambda i: (0, i, 0)),
            pl.BlockSpec((8, _BN, 16), lambda i: (0, i, 0)),
            pl.BlockSpec((_BN, D), lambda i: (i, 0)),
        ],
        out_shape=[
            jax.ShapeDtypeStruct((8, n, 16), jnp.float32),
            jax.ShapeDtypeStruct((8, n, 16), jnp.float32),
            jax.ShapeDtypeStruct((n, D), jnp.float32),
        ],
    )(x, W, b.reshape(1, D))


def _proj(x, W, b, n_split):
    n = x.shape[0]
    w = D // n_split if n_split else 0

    def body(x_ref, w_ref, b_ref, o_ref, *sp):
        y = jnp.dot(x_ref[...], w_ref[...], preferred_element_type=jnp.float32)
        y = y + b_ref[...]
        o_ref[...] = y
        for j in range(n_split):
            sp[0][j] = y[:, w * j:w * (j + 1)]

    out_specs = [pl.BlockSpec((_BN, D), lambda i: (i, 0))]
    out_shape = [jax.ShapeDtypeStruct((n, D), jnp.float32)]
    if n_split:
        out_specs.append(pl.BlockSpec((n_split, _BN, w), lambda i: (0, i, 0)))
        out_shape.append(jax.ShapeDtypeStruct((n_split, n, w), jnp.float32))

    return pl.pallas_call(
        body,
        grid=(n // _BN,),
        in_specs=[
            pl.BlockSpec((_BN, D), lambda i: (i, 0)),
            pl.BlockSpec((D, D), lambda i: (0, 0)),
            pl.BlockSpec((1, D), lambda i: (0, 0)),
        ],
        out_specs=out_specs,
        out_shape=out_shape,
    )(x, W, b.reshape(1, D))


def _combine(x, ssum, deg2, W1, W2, b, relu, n_split):
    n = x.shape[0]
    w = D // n_split if n_split else 0

    def body(x_ref, s_ref, d_ref, w1_ref, w2_ref, b_ref, o_ref, *sp):
        rdeg = 1.0 / jnp.maximum(d_ref[...], 1.0)
        m = s_ref[...] * rdeg
        y = jnp.dot(x_ref[...], w1_ref[...], preferred_element_type=jnp.float32)
        y = y + jnp.dot(m, w2_ref[...], preferred_element_type=jnp.float32)
        y = y + b_ref[...]
        if relu:
            y = jnp.maximum(y, 0.0)
        o_ref[...] = y
        for j in range(n_split):
            sp[0][j] = y[:, w * j:w * (j + 1)]

    out_specs = [pl.BlockSpec((_BN, D), lambda i: (i, 0))]
    out_shape = [jax.ShapeDtypeStruct((n, D), jnp.float32)]
    if n_split:
        out_specs.append(pl.BlockSpec((n_split, _BN, w), lambda i: (0, i, 0)))
        out_shape.append(jax.ShapeDtypeStruct((n_split, n, w), jnp.float32))

    return pl.pallas_call(
        body,
        grid=(n // _BN,),
        in_specs=[
            pl.BlockSpec((_BN, D), lambda i: (i, 0)),
            pl.BlockSpec((_BN, D), lambda i: (i, 0)),
            pl.BlockSpec((_BN, 1), lambda i: (i, 0)),
            pl.BlockSpec((D, D), lambda i: (0, 0)),
            pl.BlockSpec((D, D), lambda i: (0, 0)),
            pl.BlockSpec((1, D), lambda i: (0, 0)),
        ],
        out_specs=out_specs,
        out_shape=out_shape,
    )(x, ssum, deg2, W1, W2, b.reshape(1, D))


_NSC = 2
_NTILE = 16
_K = 7
_ZR = 256


def _segsum(tables, srcp, dstp, n_src, n_dst, w, n_chunks, with_deg):
    n_tables = len(tables)
    e_rows = srcp.shape[0]
    assert e_rows % (_NTILE * _K) == 0
    tile_rows = e_rows // _NTILE
    n_groups = tile_rows // _K
    n_passes = n_chunks // _NSC
    assert n_dst % _NTILE == 0
    wo_rows = n_dst // _NTILE
    dpt = _rup(-(-n_dst // _NTILE), 16)
    deg_n = _NTILE * dpt
    nd_acc = _rup(max(deg_n, n_dst + 1), 4096)
    zpt = nd_acc // _NTILE
    assert zpt % _ZR == 0

    mesh = plsc.VectorSubcoreMesh(core_axis_name="c", subcore_axis_name="s")

    out_type = [jax.ShapeDtypeStruct((n_chunks, n_dst, w), jnp.float32)
                for _ in range(n_tables)]
    out_type.append(jax.ShapeDtypeStruct((deg_n,), jnp.float32))

    scratch = []
    scratch += [pltpu.VMEM_SHARED((nd_acc, w), jnp.float32)
                for _ in range(n_tables)]
    scratch.append(pltpu.VMEM_SHARED((nd_acc,), jnp.float32))
    scratch.append(pltpu.VMEM((_K, 128), jnp.int32))
    scratch.append(pltpu.VMEM((_K, 128), jnp.int32))
    scratch += [pltpu.VMEM((_K, 128, w), jnp.float32)
                for _ in range(n_tables)]
    scratch.append(pltpu.VMEM((_ZR, w), jnp.float32))
    scratch.append(pltpu.VMEM((_ZR,), jnp.float32))
    scratch.append(pltpu.VMEM((128,), jnp.float32))
    scratch.append(pltpu.SemaphoreType.DMA)

    @functools.partial(pl.kernel, mesh=mesh, out_type=out_type,
                       scratch_types=scratch)
    def k(*refs):
        i = 0
        tbl = refs[i:i + n_tables]; i += n_tables
        src_hbm = refs[i]; i += 1
        dst_hbm = refs[i]; i += 1
        out = refs[i:i + n_tables]; i += n_tables
        deg_hbm = refs[i]; i += 1
        acc = refs[i:i + n_tables]; i += n_tables
        deg_acc = refs[i]; i += 1
        src_buf = refs[i]; i += 1
        dst_buf = refs[i]; i += 1
        msg = refs[i:i + n_tables]; i += n_tables
        zbuf = refs[i]; i += 1
        zdeg = refs[i]; i += 1
        ones = refs[i]; i += 1
        sem = refs[i]; i += 1

        c = lax.axis_index("c")
        s = lax.axis_index("s")

        z16 = jnp.zeros((16,), jnp.float32)

        def zrow(r, _):
            for q in range(w // 16):
                zbuf[r, pl.ds(q * 16, 16)] = z16
            return 0

        lax.fori_loop(0, _ZR, zrow, 0)
        for q in range(_ZR // 16):
            zdeg[pl.ds(q * 16, 16)] = z16
        for q in range(128 // 16):
            ones[pl.ds(q * 16, 16)] = jnp.ones((16,), jnp.float32)

        zb = s * zpt
        for t in range(n_tables):
            def zacc(g, _, _t=t):
                pltpu.sync_copy(zbuf, acc[_t].at[pl.ds(zb + g * _ZR, _ZR), :])
                return 0
            lax.fori_loop(0, zpt // _ZR, zacc, 0)
        if with_deg:
            @pl.when(c == 0)
            def _():
                def zd(g, _):
                    pltpu.sync_copy(zdeg, deg_acc.at[pl.ds(zb + g * _ZR, _ZR)])
                    return 0
                lax.fori_loop(0, zpt // _ZR, zd, 0)

        plsc.subcore_barrier()

        row0 = s * tile_rows
        for p in range(n_passes):
            j = c * n_passes + p
            off = j * n_src

            def group(g, _, _p=p, _off=off):
                rb = row0 + g * _K
                pltpu.sync_copy(src_hbm.at[pl.ds(rb, _K)], src_buf)
                pltpu.sync_copy(dst_hbm.at[pl.ds(rb, _K)], dst_buf)
                for r in range(_K):
                    for q in range(8):
                        sl = pl.ds(q * 16, 16)
                        src_buf[r, sl] = src_buf[r, sl] + _off
                cps = []
                for t in range(n_tables):
                    for r in range(_K):
                        cps.append(pltpu.async_copy(
                            tbl[t].at[src_buf.at[r]], msg[t].at[r], sem))
                for cp in cps:
                    cp.wait()
                cps = []
                for t in range(n_tables):
                    for r in range(_K):
                        cps.append(pltpu.async_copy(
                            msg[t].at[r], acc[t].at[dst_buf.at[r]], sem,
                            add=True))
                if with_deg and _p == 0:
                    @pl.when(c == 0)
                    def _():
                        dc = [pltpu.async_copy(
                            ones, deg_acc.at[dst_buf.at[r]], sem, add=True)
                            for r in range(_K)]
                        for cp in dc:
                            cp.wait()
                for cp in cps:
                    cp.wait()
                return 0

            lax.fori_loop(0, n_groups, group, 0)

        plsc.subcore_barrier()

        wb = s * wo_rows
        for p in range(n_passes):
            j = c * n_passes + p
            for t in range(n_tables):
                pltpu.sync_copy(acc[t].at[pl.ds(wb, wo_rows), :],
                                out[t].at[j, pl.ds(wb, wo_rows), :])
        if with_deg:
            @pl.when(c == 0)
            def _():
                db = s * dpt
                pltpu.sync_copy(deg_acc.at[pl.ds(db, dpt)],
                                deg_hbm.at[pl.ds(db, dpt)])

    return k(*tables, srcp, dstp)


def _pad_edges(src, dst, n_src, n_dst):
    e = src.shape[0]
    e_pad = _rup(e, _NTILE * _K * 128)
    pad = e_pad - e
    psrc = jnp.arange(pad, dtype=jnp.int32) % n_src
    pdst = jnp.full((pad,), n_dst, jnp.int32)
    srcp = jnp.concatenate([src, psrc]).reshape(-1, 128)
    dstp = jnp.concatenate([dst, pdst]).reshape(-1, 128)
    return srcp, dstp


def _tr(x):
    return jnp.moveaxis(x, 0, 1).reshape(x.shape[1], D)


def kernel(student_x, concept_x, lecture_x, und_src, und_dst, tea_src,
           tea_dst, W_fs, b_fs, W_fc, b_fc, W_fl, b_fl,
           Wu_self, Wu_neigh, bu, Wt_self, Wt_neigh, bt):
    NS, NC, NL = student_x.shape[0], concept_x.shape[0], lecture_x.shape[0]

    s_split, r_split, s_out = _proj_student(student_x, W_fs, b_fs)
    c0, c0_split = _proj(concept_x, W_fc, b_fc, n_split=2)
    l0 = _proj(lecture_x, W_fl, b_fl, n_split=0)[0]

    und_srcp, und_dstp = _pad_edges(und_src, und_dst, NS, NC)
    tea_srcp, tea_dstp = _pad_edges(tea_src, tea_dst, NC, NL)

    U0c, U1c, degc_raw = _segsum(
        [s_split.reshape(-1, 16), r_split.reshape(-1, 16)],
        und_srcp, und_dstp, NS, NC, w=16, n_chunks=8, with_deg=True)
    U0, U1 = _tr(U0c), _tr(U1c)
    degc = degc_raw[:NC].reshape(NC, 1)

    T0c, degl_raw = _segsum([c0_split.reshape(-1, 64)],
                            tea_srcp, tea_dstp, NC, NL, w=64, n_chunks=2,
                            with_deg=True)
    degl = degl_raw[:NL].reshape(NL, 1)

    c1, c1_split = _combine(c0, U0, degc, Wu_self[0], Wu_neigh[0], bu[0],
                            relu=True, n_split=2)
    l1 = _combine(l0, _tr(T0c), degl, Wt_self[0], Wt_neigh[0], bt[0],
                  relu=True, n_split=0)[0]

    T1c, _ = _segsum([c1_split.reshape(-1, 64)], tea_srcp, tea_dstp, NC, NL,
                     w=64, n_chunks=2, with_deg=False)
    c2, c2_split = _combine(c1, U1, degc, Wu_self[1], Wu_neigh[1], bu[1],
                            relu=True, n_split=2)
    l2 = _combine(l1, _tr(T1c), degl, Wt_self[1], Wt_neigh[1], bt[1],
                  relu=True, n_split=0)[0]

    T2c, _ = _segsum([c2_split.reshape(-1, 64)], tea_srcp, tea_dstp, NC, NL,
                     w=64, n_chunks=2, with_deg=False)
    c3 = _combine(c2, U1, degc, Wu_self[2], Wu_neigh[2], bu[2],
                  relu=False, n_split=0)[0]
    l3 = _combine(l2, _tr(T2c), degl, Wt_self[2], Wt_neigh[2], bt[2],
                  relu=False, n_split=0)[0]

    return (s_out, c3, l3)


# speedup vs baseline: 3.7407x; 3.7407x over previous
import functools

import jax
import jax.numpy as jnp
from jax import lax
from jax.experimental import pallas as pl
from jax.experimental.pallas import tpu as pltpu
from jax.experimental.pallas import tpu_sc as plsc

D = 128
_BN = 2000


def _rup(x, m):
    return (x + m - 1) // m * m


def _proj_student(x, W, b):
    n = x.shape[0]

    def body(x_ref, w_ref, b_ref, ys_ref, rs_ref, rd_ref):
        y = jnp.dot(x_ref[...], w_ref[...], preferred_element_type=jnp.float32)
        y = y + b_ref[...]
        ry = jnp.maximum(y, 0.0)
        rd_ref[...] = ry
        for j in range(8):
            ys_ref[j] = y[:, 16 * j:16 * (j + 1)]
            rs_ref[j] = ry[:, 16 * j:16 * (j + 1)]

    return pl.pallas_call(
        body,
        grid=(n // _BN,),
        in_specs=[
            pl.BlockSpec((_BN, D), lambda i: (i, 0)),
            pl.BlockSpec((D, D), lambda i: (0, 0)),
            pl.BlockSpec((1, D), lambda i: (0, 0)),
        ],
        out_specs=[
            pl.BlockSpec((8, _BN, 16), lambda i: (0, i, 0)),
            pl.BlockSpec((8, _BN, 16), lambda i: (0, i, 0)),
            pl.BlockSpec((_BN, D), lambda i: (i, 0)),
        ],
        out_shape=[
            jax.ShapeDtypeStruct((8, n, 16), jnp.float32),
            jax.ShapeDtypeStruct((8, n, 16), jnp.float32),
            jax.ShapeDtypeStruct((n, D), jnp.float32),
        ],
    )(x, W, b.reshape(1, D))


def _proj(x, W, b, n_split):
    n = x.shape[0]
    w = D // n_split if n_split else 0

    def body(x_ref, w_ref, b_ref, o_ref, *sp):
        y = jnp.dot(x_ref[...], w_ref[...], preferred_element_type=jnp.float32)
        y = y + b_ref[...]
        o_ref[...] = y
        for j in range(n_split):
            sp[0][j] = y[:, w * j:w * (j + 1)]

    out_specs = [pl.BlockSpec((_BN, D), lambda i: (i, 0))]
    out_shape = [jax.ShapeDtypeStruct((n, D), jnp.float32)]
    if n_split:
        out_specs.append(pl.BlockSpec((n_split, _BN, w), lambda i: (0, i, 0)))
        out_shape.append(jax.ShapeDtypeStruct((n_split, n, w), jnp.float32))

    return pl.pallas_call(
        body,
        grid=(n // _BN,),
        in_specs=[
            pl.BlockSpec((_BN, D), lambda i: (i, 0)),
            pl.BlockSpec((D, D), lambda i: (0, 0)),
            pl.BlockSpec((1, D), lambda i: (0, 0)),
        ],
        out_specs=out_specs,
        out_shape=out_shape,
    )(x, W, b.reshape(1, D))


def _combine(x, ssum, deg2, W1, W2, b, relu, n_split):
    n = x.shape[0]
    w = D // n_split if n_split else 0

    def body(x_ref, s_ref, d_ref, w1_ref, w2_ref, b_ref, o_ref, *sp):
        rdeg = 1.0 / jnp.maximum(d_ref[...], 1.0)
        m = s_ref[...] * rdeg
        y = jnp.dot(x_ref[...], w1_ref[...], preferred_element_type=jnp.float32)
        y = y + jnp.dot(m, w2_ref[...], preferred_element_type=jnp.float32)
        y = y + b_ref[...]
        if relu:
            y = jnp.maximum(y, 0.0)
        o_ref[...] = y
        for j in range(n_split):
            sp[0][j] = y[:, w * j:w * (j + 1)]

    out_specs = [pl.BlockSpec((_BN, D), lambda i: (i, 0))]
    out_shape = [jax.ShapeDtypeStruct((n, D), jnp.float32)]
    if n_split:
        out_specs.append(pl.BlockSpec((n_split, _BN, w), lambda i: (0, i, 0)))
        out_shape.append(jax.ShapeDtypeStruct((n_split, n, w), jnp.float32))

    return pl.pallas_call(
        body,
        grid=(n // _BN,),
        in_specs=[
            pl.BlockSpec((_BN, D), lambda i: (i, 0)),
            pl.BlockSpec((_BN, D), lambda i: (i, 0)),
            pl.BlockSpec((_BN, 1), lambda i: (i, 0)),
            pl.BlockSpec((D, D), lambda i: (0, 0)),
            pl.BlockSpec((D, D), lambda i: (0, 0)),
            pl.BlockSpec((1, D), lambda i: (0, 0)),
        ],
        out_specs=out_specs,
        out_shape=out_shape,
    )(x, ssum, deg2, W1, W2, b.reshape(1, D))


_NSC = 2
_NTILE = 16
_K = 8
_ZR = 64


def _segsum(tables, srcp, dstp, n_src, n_dst, w, n_chunks, with_deg):
    n_tables = len(tables)
    e_rows = srcp.shape[1]
    assert e_rows % (_NTILE * _K) == 0
    tile_rows = e_rows // _NTILE
    n_groups = tile_rows // _K
    n_passes = n_chunks // _NSC
    nd_out = _rup(n_dst, _NTILE * 8)
    wo_rows = nd_out // _NTILE
    dpt = _rup(-(-n_dst // _NTILE), 16)
    deg_n = _NTILE * dpt
    nd_acc = _rup(max(deg_n, nd_out, n_dst + 1), _NTILE * _ZR)
    zpt = nd_acc // _NTILE
    assert zpt % _ZR == 0

    mesh = plsc.VectorSubcoreMesh(core_axis_name="c", subcore_axis_name="s",
                                  num_cores=_NSC, num_subcores=_NTILE)

    out_type = [jax.ShapeDtypeStruct((n_chunks, nd_out, w), jnp.float32)
                for _ in range(n_tables)]
    out_type.append(jax.ShapeDtypeStruct((deg_n,), jnp.float32))

    scratch = []
    scratch += [pltpu.VMEM_SHARED((nd_acc, w), jnp.float32)
                for _ in range(n_tables)]
    scratch.append(pltpu.VMEM_SHARED((nd_acc,), jnp.float32))
    scratch.append(pltpu.VMEM((_K, 128), jnp.int32))
    scratch.append(pltpu.VMEM((_K, 128), jnp.int32))
    scratch.append(pltpu.VMEM((_K, 128, w), jnp.float32))
    scratch.append(pltpu.VMEM((_ZR, w), jnp.float32))
    scratch.append(pltpu.VMEM((_ZR,), jnp.float32))
    scratch.append(pltpu.VMEM((128,), jnp.float32))
    scratch.append(pltpu.SemaphoreType.DMA)

    @functools.partial(
        pl.kernel, mesh=mesh, out_type=out_type, scratch_types=scratch,
        compiler_params=pltpu.CompilerParams(use_tc_tiling_on_sc=False))
    def k(*refs):
        i = 0
        tbl = refs[i:i + n_tables]; i += n_tables
        src_hbm = refs[i]; i += 1
        dst_hbm = refs[i]; i += 1
        out = refs[i:i + n_tables]; i += n_tables
        deg_hbm = refs[i]; i += 1
        acc = refs[i:i + n_tables]; i += n_tables
        deg_acc = refs[i]; i += 1
        src_buf = refs[i]; i += 1
        dst_buf = refs[i]; i += 1
        msg = refs[i]; i += 1
        zbuf = refs[i]; i += 1
        zdeg = refs[i]; i += 1
        ones = refs[i]; i += 1
        sem = refs[i]; i += 1

        c = lax.axis_index("c")
        s = lax.axis_index("s")

        z16 = jnp.zeros((16,), jnp.float32)

        def zrow(r, _):
            for q in range(w // 16):
                zbuf[r, pl.ds(q * 16, 16)] = z16
            return 0

        lax.fori_loop(0, _ZR, zrow, 0)
        for q in range(_ZR // 16):
            zdeg[pl.ds(q * 16, 16)] = z16
        for q in range(128 // 16):
            ones[pl.ds(q * 16, 16)] = jnp.ones((16,), jnp.float32)

        zb = s * zpt

        def zero_accs():
            for t in range(n_tables):
                def zacc(g, _, _t=t):
                    pltpu.sync_copy(zbuf,
                                    acc[_t].at[pl.ds(zb + g * _ZR, _ZR), :])
                    return 0
                lax.fori_loop(0, zpt // _ZR, zacc, 0)

        zero_accs()
        if with_deg:
            @pl.when(c == 0)
            def _():
                def zd(g, _):
                    pltpu.sync_copy(zdeg, deg_acc.at[pl.ds(zb + g * _ZR, _ZR)])
                    return 0
                lax.fori_loop(0, zpt // _ZR, zd, 0)

        plsc.subcore_barrier()

        row0 = s * tile_rows
        wb = s * wo_rows
        for p in range(n_passes):
            j = c * n_passes + p

            def group(g, _, _p=p, _j=j):
                rb = row0 + g * _K
                pltpu.sync_copy(src_hbm.at[_j, pl.ds(rb, _K)], src_buf)
                pltpu.sync_copy(dst_hbm.at[pl.ds(rb, _K)], dst_buf)
                for t in range(n_tables):
                    cps = [pltpu.async_copy(
                        tbl[t].at[src_buf.at[r]], msg.at[r], sem)
                        for r in range(_K)]
                    for cp in cps:
                        cp.wait()
                    cps = [pltpu.async_copy(
                        msg.at[r], acc[t].at[dst_buf.at[r]], sem, add=True)
                        for r in range(_K)]
                    for cp in cps:
                        cp.wait()
                if with_deg and _p == 0:
                    @pl.when(c == 0)
                    def _():
                        dc = [pltpu.async_copy(
                            ones, deg_acc.at[dst_buf.at[r]], sem, add=True)
                            for r in range(_K)]
                        for cp in dc:
                            cp.wait()
                return 0

            lax.fori_loop(0, n_groups, group, 0)
            plsc.subcore_barrier()
            for t in range(n_tables):
                pltpu.sync_copy(acc[t].at[pl.ds(wb, wo_rows), :],
                                out[t].at[j, pl.ds(wb, wo_rows), :])
            if p != n_passes - 1:
                plsc.subcore_barrier()
                zero_accs()
                plsc.subcore_barrier()

        if with_deg:
            @pl.when(c == 0)
            def _():
                db = s * dpt
                pltpu.sync_copy(deg_acc.at[pl.ds(db, dpt)],
                                deg_hbm.at[pl.ds(db, dpt)])

    res = k(*tables, srcp, dstp)
    return tuple(o[:, :n_dst, :] for o in res[:-1]) + (res[-1][:n_dst],)


def _pad_edges(src, dst, n_src, n_dst, n_chunks):
    e = src.shape[0]
    e_pad = _rup(e, _NTILE * _K * 128)
    pad = e_pad - e
    psrc = jnp.arange(pad, dtype=jnp.int32) % n_src
    pdst = jnp.full((pad,), n_dst, jnp.int32)
    srcp = jnp.concatenate([src, psrc]).reshape(-1, 128)
    offs = (jnp.arange(n_chunks, dtype=jnp.int32) * n_src)[:, None, None]
    srcp = srcp[None] + offs
    dstp = jnp.concatenate([dst, pdst]).reshape(-1, 128)
    return srcp, dstp


def _tr(x):
    return jnp.moveaxis(x, 0, 1).reshape(x.shape[1], D)


def kernel(student_x, concept_x, lecture_x, und_src, und_dst, tea_src,
           tea_dst, W_fs, b_fs, W_fc, b_fc, W_fl, b_fl,
           Wu_self, Wu_neigh, bu, Wt_self, Wt_neigh, bt):
    NS, NC, NL = student_x.shape[0], concept_x.shape[0], lecture_x.shape[0]

    s_split, r_split, s_out = _proj_student(student_x, W_fs, b_fs)
    c0, c0_split = _proj(concept_x, W_fc, b_fc, n_split=2)
    l0 = _proj(lecture_x, W_fl, b_fl, n_split=0)[0]

    und_srcp, und_dstp = _pad_edges(und_src, und_dst, NS, NC, n_chunks=8)
    tea_srcp, tea_dstp = _pad_edges(tea_src, tea_dst, NC, NL, n_chunks=2)

    U0c, U1c, degc_raw = _segsum(
        [s_split.reshape(-1, 16), r_split.reshape(-1, 16)],
        und_srcp, und_dstp, NS, NC, w=16, n_chunks=8, with_deg=True)
    U0, U1 = _tr(U0c), _tr(U1c)
    degc = degc_raw.reshape(NC, 1)

    T0c, degl_raw = _segsum([c0_split.reshape(-1, 64)],
                            tea_srcp, tea_dstp, NC, NL, w=64, n_chunks=2,
                            with_deg=True)
    degl = degl_raw.reshape(NL, 1)

    c1, c1_split = _combine(c0, U0, degc, Wu_self[0], Wu_neigh[0], bu[0],
                            relu=True, n_split=2)
    l1 = _combine(l0, _tr(T0c), degl, Wt_self[0], Wt_neigh[0], bt[0],
                  relu=True, n_split=0)[0]

    T1c, _ = _segsum([c1_split.reshape(-1, 64)], tea_srcp, tea_dstp, NC, NL,
                     w=64, n_chunks=2, with_deg=False)
    c2, c2_split = _combine(c1, U1, degc, Wu_self[1], Wu_neigh[1], bu[1],
                            relu=True, n_split=2)
    l2 = _combine(l1, _tr(T1c), degl, Wt_self[1], Wt_neigh[1], bt[1],
                  relu=True, n_split=0)[0]

    T2c, _ = _segsum([c2_split.reshape(-1, 64)], tea_srcp, tea_dstp, NC, NL,
                     w=64, n_chunks=2, with_deg=False)
    c3 = _combine(c2, U1, degc, Wu_self[2], Wu_neigh[2], bu[2],
                  relu=False, n_split=0)[0]
    l3 = _combine(l2, _tr(T2c), degl, Wt_self[2], Wt_neigh[2], bt[2],
                  relu=False, n_split=0)[0]

    return (s_out, c3, l3)


# speedup vs baseline: 5.7936x; 1.5488x over previous
import functools

import jax
import jax.numpy as jnp
from jax import lax
from jax.experimental import pallas as pl
from jax.experimental.pallas import tpu as pltpu
from jax.experimental.pallas import tpu_sc as plsc

D = 128
_BN = 2000


def _rup(x, m):
    return (x + m - 1) // m * m


def _proj(x, W, b, with_relu):
    n = x.shape[0]

    def body(x_ref, w_ref, b_ref, *out_refs):
        y = jnp.dot(x_ref[...], w_ref[...], preferred_element_type=jnp.float32)
        y = y + b_ref[...]
        out_refs[0][...] = y
        if with_relu:
            out_refs[1][...] = jnp.maximum(y, 0.0)

    n_out = 2 if with_relu else 1
    return pl.pallas_call(
        body,
        grid=(n // _BN,),
        in_specs=[
            pl.BlockSpec((_BN, D), lambda i: (i, 0)),
            pl.BlockSpec((D, D), lambda i: (0, 0)),
            pl.BlockSpec((1, D), lambda i: (0, 0)),
        ],
        out_specs=[pl.BlockSpec((_BN, D), lambda i: (i, 0))] * n_out,
        out_shape=[jax.ShapeDtypeStruct((n, D), jnp.float32)] * n_out,
    )(x, W, b.reshape(1, D))


def _combine(x, ssum_c, deg2, W1, W2, b, relu):
    n = x.shape[0]
    k, _, w = ssum_c.shape

    def body(x_ref, s_ref, d_ref, w1_ref, w2_ref, b_ref, o_ref):
        rdeg = 1.0 / jnp.maximum(d_ref[...], 1.0)
        m = jnp.concatenate([s_ref[j] for j in range(k)], axis=-1) * rdeg
        y = jnp.dot(x_ref[...], w1_ref[...], preferred_element_type=jnp.float32)
        y = y + jnp.dot(m, w2_ref[...], preferred_element_type=jnp.float32)
        y = y + b_ref[...]
        if relu:
            y = jnp.maximum(y, 0.0)
        o_ref[...] = y

    return pl.pallas_call(
        body,
        grid=(n // _BN,),
        in_specs=[
            pl.BlockSpec((_BN, D), lambda i: (i, 0)),
            pl.BlockSpec((k, _BN, w), lambda i: (0, i, 0)),
            pl.BlockSpec((_BN, 1), lambda i: (i, 0)),
            pl.BlockSpec((D, D), lambda i: (0, 0)),
            pl.BlockSpec((D, D), lambda i: (0, 0)),
            pl.BlockSpec((1, D), lambda i: (0, 0)),
        ],
        out_specs=pl.BlockSpec((_BN, D), lambda i: (i, 0)),
        out_shape=jax.ShapeDtypeStruct((n, D), jnp.float32),
    )(x, ssum_c, deg2, W1, W2, b.reshape(1, D))


_NSC = 2
_NTILE = 16
_K = 8
_ZR = 64


def _segsum(tables, srcp, dstp, n_src, n_dst, w, n_chunks, with_deg):
    n_tables = len(tables)
    e_rows = srcp.shape[1]
    assert e_rows % (_NTILE * _K) == 0
    tile_rows = e_rows // _NTILE
    n_groups = tile_rows // _K
    n_passes = n_chunks // _NSC
    nd_out = _rup(n_dst, _NTILE * 8)
    wo_rows = nd_out // _NTILE
    dpt = _rup(-(-n_dst // _NTILE), 16)
    deg_n = _NTILE * dpt
    nd_acc = _rup(max(deg_n, nd_out, n_dst + 1), _NTILE * _ZR)
    zpt = nd_acc // _NTILE
    assert zpt % _ZR == 0

    mesh = plsc.VectorSubcoreMesh(core_axis_name="c", subcore_axis_name="s",
                                  num_cores=_NSC, num_subcores=_NTILE)

    out_type = [jax.ShapeDtypeStruct((n_chunks, nd_out, w), jnp.float32)
                for _ in range(n_tables)]
    out_type.append(jax.ShapeDtypeStruct((deg_n,), jnp.float32))

    scratch = []
    scratch += [pltpu.VMEM_SHARED((nd_acc, w), jnp.float32)
                for _ in range(n_tables)]
    scratch.append(pltpu.VMEM_SHARED((nd_acc,), jnp.float32))
    scratch.append(pltpu.VMEM((_K, 128), jnp.int32))
    scratch.append(pltpu.VMEM((_K, 128), jnp.int32))
    scratch.append(pltpu.VMEM((_K, 128, w), jnp.float32))
    scratch.append(pltpu.VMEM((_ZR, w), jnp.float32))
    scratch.append(pltpu.VMEM((_ZR,), jnp.float32))
    scratch.append(pltpu.VMEM((128,), jnp.float32))
    scratch.append(pltpu.SemaphoreType.DMA)

    @functools.partial(
        pl.kernel, mesh=mesh, out_type=out_type, scratch_types=scratch,
        compiler_params=pltpu.CompilerParams(use_tc_tiling_on_sc=False))
    def k(*refs):
        i = 0
        tbl = refs[i:i + n_tables]; i += n_tables
        src_hbm = refs[i]; i += 1
        dst_hbm = refs[i]; i += 1
        out = refs[i:i + n_tables]; i += n_tables
        deg_hbm = refs[i]; i += 1
        acc = refs[i:i + n_tables]; i += n_tables
        deg_acc = refs[i]; i += 1
        src_buf = refs[i]; i += 1
        dst_buf = refs[i]; i += 1
        msg = refs[i]; i += 1
        zbuf = refs[i]; i += 1
        zdeg = refs[i]; i += 1
        ones = refs[i]; i += 1
        sem = refs[i]; i += 1

        c = lax.axis_index("c")
        s = lax.axis_index("s")

        z16 = jnp.zeros((16,), jnp.float32)

        def zrow(r, _):
            for q in range(w // 16):
                zbuf[r, pl.ds(q * 16, 16)] = z16
            return 0

        lax.fori_loop(0, _ZR, zrow, 0)
        for q in range(_ZR // 16):
            zdeg[pl.ds(q * 16, 16)] = z16
        for q in range(128 // 16):
            ones[pl.ds(q * 16, 16)] = jnp.ones((16,), jnp.float32)

        zb = s * zpt

        def zero_accs():
            for t in range(n_tables):
                def zacc(g, _, _t=t):
                    pltpu.sync_copy(zbuf,
                                    acc[_t].at[pl.ds(zb + g * _ZR, _ZR), :])
                    return 0
                lax.fori_loop(0, zpt // _ZR, zacc, 0)

        zero_accs()
        if with_deg:
            @pl.when(c == 0)
            def _():
                def zd(g, _):
                    pltpu.sync_copy(zdeg, deg_acc.at[pl.ds(zb + g * _ZR, _ZR)])
                    return 0
                lax.fori_loop(0, zpt // _ZR, zd, 0)

        plsc.subcore_barrier()

        row0 = s * tile_rows
        wb = s * wo_rows
        for p in range(n_passes):
            j = c * n_passes + p

            def group(g, _, _p=p, _j=j):
                rb = row0 + g * _K
                pltpu.sync_copy(src_hbm.at[_j, pl.ds(rb, _K)], src_buf)
                pltpu.sync_copy(dst_hbm.at[pl.ds(rb, _K)], dst_buf)
                for t in range(n_tables):
                    cps = [pltpu.async_copy(
                        tbl[t].at[src_buf.at[r]], msg.at[r], sem)
                        for r in range(_K)]
                    for cp in cps:
                        cp.wait()
                    cps = [pltpu.async_copy(
                        msg.at[r], acc[t].at[dst_buf.at[r]], sem, add=True)
                        for r in range(_K)]
                    for cp in cps:
                        cp.wait()
                if with_deg and _p == 0:
                    @pl.when(c == 0)
                    def _():
                        dc = [pltpu.async_copy(
                            ones, deg_acc.at[dst_buf.at[r]], sem, add=True)
                            for r in range(_K)]
                        for cp in dc:
                            cp.wait()
                return 0

            lax.fori_loop(0, n_groups, group, 0)
            plsc.subcore_barrier()
            for t in range(n_tables):
                pltpu.sync_copy(acc[t].at[pl.ds(wb, wo_rows), :],
                                out[t].at[j, pl.ds(wb, wo_rows), :])
            if p != n_passes - 1:
                plsc.subcore_barrier()
                zero_accs()
                plsc.subcore_barrier()

        if with_deg:
            @pl.when(c == 0)
            def _():
                db = s * dpt
                pltpu.sync_copy(deg_acc.at[pl.ds(db, dpt)],
                                deg_hbm.at[pl.ds(db, dpt)])

    res = k(*tables, srcp, dstp)
    return tuple(o[:, :n_dst, :] for o in res[:-1]) + (res[-1][:n_dst],)


def _pad_edges(src, dst, n_src, n_dst, n_chunks):
    e = src.shape[0]
    e_pad = _rup(e, _NTILE * _K * 128)
    pad = e_pad - e
    psrc = jnp.arange(pad, dtype=jnp.int32) % n_src
    pdst = jnp.full((pad,), n_dst, jnp.int32)
    srcp = jnp.concatenate([src, psrc]).reshape(-1, 128)
    offs = jnp.arange(n_chunks, dtype=jnp.int32)[:, None, None]
    srcp = srcp[None] * n_chunks + offs
    dstp = jnp.concatenate([dst, pdst]).reshape(-1, 128)
    return srcp, dstp


def kernel(student_x, concept_x, lecture_x, und_src, und_dst, tea_src,
           tea_dst, W_fs, b_fs, W_fc, b_fc, W_fl, b_fl,
           Wu_self, Wu_neigh, bu, Wt_self, Wt_neigh, bt):
    NS, NC, NL = student_x.shape[0], concept_x.shape[0], lecture_x.shape[0]

    s0, s_out = _proj(student_x, W_fs, b_fs, with_relu=True)
    c0 = _proj(concept_x, W_fc, b_fc, with_relu=False)[0]
    l0 = _proj(lecture_x, W_fl, b_fl, with_relu=False)[0]

    und_srcp, und_dstp = _pad_edges(und_src, und_dst, NS, NC, n_chunks=8)
    tea_srcp, tea_dstp = _pad_edges(tea_src, tea_dst, NC, NL, n_chunks=2)

    U0c, U1c, degc_raw = _segsum(
        [s0.reshape(-1, 16), s_out.reshape(-1, 16)],
        und_srcp, und_dstp, NS, NC, w=16, n_chunks=8, with_deg=True)
    degc = degc_raw.reshape(NC, 1)

    T0c, degl_raw = _segsum([c0.reshape(-1, 64)],
                            tea_srcp, tea_dstp, NC, NL, w=64, n_chunks=2,
                            with_deg=True)
    degl = degl_raw.reshape(NL, 1)

    c1 = _combine(c0, U0c, degc, Wu_self[0], Wu_neigh[0], bu[0], relu=True)
    l1 = _combine(l0, T0c, degl, Wt_self[0], Wt_neigh[0], bt[0], relu=True)

    T1c, _ = _segsum([c1.reshape(-1, 64)], tea_srcp, tea_dstp, NC, NL,
                     w=64, n_chunks=2, with_deg=False)
    c2 = _combine(c1, U1c, degc, Wu_self[1], Wu_neigh[1], bu[1], relu=True)
    l2 = _combine(l1, T1c, degl, Wt_self[1], Wt_neigh[1], bt[1], relu=True)

    T2c, _ = _segsum([c2.reshape(-1, 64)], tea_srcp, tea_dstp, NC, NL,
                     w=64, n_chunks=2, with_deg=False)
    c3 = _combine(c2, U1c, degc, Wu_self[2], Wu_neigh[2], bu[2], relu=False)
    l3 = _combine(l2, T2c, degl, Wt_self[2], Wt_neigh[2], bt[2], relu=False)

    return (s_out, c3, l3)


# speedup vs baseline: 6.0083x; 1.0371x over previous
import functools

import jax
import jax.numpy as jnp
from jax import lax
from jax.experimental import pallas as pl
from jax.experimental.pallas import tpu as pltpu
from jax.experimental.pallas import tpu_sc as plsc

D = 128
_BN = 2000


def _rup(x, m):
    return (x + m - 1) // m * m


def _proj(x, W, b, with_relu):
    n = x.shape[0]

    def body(x_ref, w_ref, b_ref, *out_refs):
        y = jnp.dot(x_ref[...], w_ref[...], preferred_element_type=jnp.float32)
        y = y + b_ref[...]
        out_refs[0][...] = y
        if with_relu:
            out_refs[1][...] = jnp.maximum(y, 0.0)

    n_out = 2 if with_relu else 1
    return pl.pallas_call(
        body,
        grid=(n // _BN,),
        in_specs=[
            pl.BlockSpec((_BN, D), lambda i: (i, 0)),
            pl.BlockSpec((D, D), lambda i: (0, 0)),
            pl.BlockSpec((1, D), lambda i: (0, 0)),
        ],
        out_specs=[pl.BlockSpec((_BN, D), lambda i: (i, 0))] * n_out,
        out_shape=[jax.ShapeDtypeStruct((n, D), jnp.float32)] * n_out,
    )(x, W, b.reshape(1, D))


def _combine(x, ssum, deg2, W1, W2, b, relu):
    n = x.shape[0]

    def body(x_ref, s_ref, d_ref, w1_ref, w2_ref, b_ref, o_ref):
        rdeg = 1.0 / jnp.maximum(d_ref[...], 1.0)
        m = s_ref[...] * rdeg
        y = jnp.dot(x_ref[...], w1_ref[...], preferred_element_type=jnp.float32)
        y = y + jnp.dot(m, w2_ref[...], preferred_element_type=jnp.float32)
        y = y + b_ref[...]
        if relu:
            y = jnp.maximum(y, 0.0)
        o_ref[...] = y

    return pl.pallas_call(
        body,
        grid=(n // _BN,),
        in_specs=[
            pl.BlockSpec((_BN, D), lambda i: (i, 0)),
            pl.BlockSpec((_BN, D), lambda i: (i, 0)),
            pl.BlockSpec((_BN, 1), lambda i: (i, 0)),
            pl.BlockSpec((D, D), lambda i: (0, 0)),
            pl.BlockSpec((D, D), lambda i: (0, 0)),
            pl.BlockSpec((1, D), lambda i: (0, 0)),
        ],
        out_specs=pl.BlockSpec((_BN, D), lambda i: (i, 0)),
        out_shape=jax.ShapeDtypeStruct((n, D), jnp.float32),
    )(x, ssum, deg2, W1, W2, b.reshape(1, D))


_NSC = 2
_NTILE = 16
_K = 8
_ZR = 64


def _segsum(tables, srcp, dstp, n_src, n_dst, w, n_chunks, with_deg):
    n_tables = len(tables)
    e_rows = srcp.shape[1]
    assert e_rows % (_NTILE * _K) == 0
    tile_rows = e_rows // _NTILE
    n_groups = tile_rows // _K
    n_passes = n_chunks // _NSC
    nd_out = _rup(n_dst, _NTILE)
    wo_rows = nd_out // _NTILE
    dpt = _rup(-(-n_dst // _NTILE), 16)
    deg_n = _NTILE * dpt
    nd_acc = _rup(max(deg_n, nd_out, n_dst + 1), _NTILE * _ZR)
    zpt = nd_acc // _NTILE
    assert zpt % _ZR == 0

    mesh = plsc.VectorSubcoreMesh(core_axis_name="c", subcore_axis_name="s",
                                  num_cores=_NSC, num_subcores=_NTILE)

    out_type = [jax.ShapeDtypeStruct((nd_out, n_chunks, w), jnp.float32)
                for _ in range(n_tables)]
    out_type.append(jax.ShapeDtypeStruct((deg_n,), jnp.float32))

    scratch = []
    scratch += [pltpu.VMEM_SHARED((nd_acc, w), jnp.float32)
                for _ in range(n_tables)]
    scratch.append(pltpu.VMEM_SHARED((nd_acc,), jnp.float32))
    scratch.append(pltpu.VMEM((_K, 128), jnp.int32))
    scratch.append(pltpu.VMEM((_K, 128), jnp.int32))
    scratch.append(pltpu.VMEM((_K, 128, w), jnp.float32))
    scratch.append(pltpu.VMEM((_ZR, w), jnp.float32))
    scratch.append(pltpu.VMEM((_ZR,), jnp.float32))
    scratch.append(pltpu.VMEM((128,), jnp.float32))
    scratch.append(pltpu.SemaphoreType.DMA)

    @functools.partial(
        pl.kernel, mesh=mesh, out_type=out_type, scratch_types=scratch,
        compiler_params=pltpu.CompilerParams(use_tc_tiling_on_sc=False))
    def k(*refs):
        i = 0
        tbl = refs[i:i + n_tables]; i += n_tables
        src_hbm = refs[i]; i += 1
        dst_hbm = refs[i]; i += 1
        out = refs[i:i + n_tables]; i += n_tables
        deg_hbm = refs[i]; i += 1
        acc = refs[i:i + n_tables]; i += n_tables
        deg_acc = refs[i]; i += 1
        src_buf = refs[i]; i += 1
        dst_buf = refs[i]; i += 1
        msg = refs[i]; i += 1
        zbuf = refs[i]; i += 1
        zdeg = refs[i]; i += 1
        ones = refs[i]; i += 1
        sem = refs[i]; i += 1

        c = lax.axis_index("c")
        s = lax.axis_index("s")

        z16 = jnp.zeros((16,), jnp.float32)

        def zrow(r, _):
            for q in range(w // 16):
                zbuf[r, pl.ds(q * 16, 16)] = z16
            return 0

        lax.fori_loop(0, _ZR, zrow, 0)
        for q in range(_ZR // 16):
            zdeg[pl.ds(q * 16, 16)] = z16
        for q in range(128 // 16):
            ones[pl.ds(q * 16, 16)] = jnp.ones((16,), jnp.float32)

        zb = s * zpt

        def zero_accs():
            for t in range(n_tables):
                def zacc(g, _, _t=t):
                    pltpu.sync_copy(zbuf,
                                    acc[_t].at[pl.ds(zb + g * _ZR, _ZR), :])
                    return 0
                lax.fori_loop(0, zpt // _ZR, zacc, 0)

        zero_accs()
        if with_deg:
            @pl.when(c == 0)
            def _():
                def zd(g, _):
                    pltpu.sync_copy(zdeg, deg_acc.at[pl.ds(zb + g * _ZR, _ZR)])
                    return 0
                lax.fori_loop(0, zpt // _ZR, zd, 0)

        plsc.subcore_barrier()

        row0 = s * tile_rows
        wb = s * wo_rows
        for p in range(n_passes):
            j = c * n_passes + p

            def group(g, _, _p=p, _j=j):
                rb = row0 + g * _K
                pltpu.sync_copy(src_hbm.at[_j, pl.ds(rb, _K)], src_buf)
                pltpu.sync_copy(dst_hbm.at[pl.ds(rb, _K)], dst_buf)
                for t in range(n_tables):
                    cps = [pltpu.async_copy(
                        tbl[t].at[src_buf.at[r]], msg.at[r], sem)
                        for r in range(_K)]
                    for cp in cps:
                        cp.wait()
                    cps = [pltpu.async_copy(
                        msg.at[r], acc[t].at[dst_buf.at[r]], sem, add=True)
                        for r in range(_K)]
                    for cp in cps:
                        cp.wait()
                if with_deg and _p == 0:
                    @pl.when(c == 0)
                    def _():
                        dc = [pltpu.async_copy(
                            ones, deg_acc.at[dst_buf.at[r]], sem, add=True)
                            for r in range(_K)]
                        for cp in dc:
                            cp.wait()
                return 0

            lax.fori_loop(0, n_groups, group, 0)
            plsc.subcore_barrier()
            for t in range(n_tables):
                pltpu.sync_copy(acc[t].at[pl.ds(wb, wo_rows), :],
                                out[t].at[pl.ds(wb, wo_rows), j, :])
            if p != n_passes - 1:
                plsc.subcore_barrier()
                zero_accs()
                plsc.subcore_barrier()

        if with_deg:
            @pl.when(c == 0)
            def _():
                db = s * dpt
                pltpu.sync_copy(deg_acc.at[pl.ds(db, dpt)],
                                deg_hbm.at[pl.ds(db, dpt)])

    res = k(*tables, srcp, dstp)
    sums = tuple(o[:n_dst].reshape(n_dst, n_chunks * w) for o in res[:-1])
    return sums + (res[-1][:n_dst],)


def _pad_edges(src, dst, n_src, n_dst, n_chunks):
    e = src.shape[0]
    e_pad = _rup(e, _NTILE * _K * 128)
    pad = e_pad - e
    psrc = jnp.arange(pad, dtype=jnp.int32) % n_src
    pdst = jnp.full((pad,), n_dst, jnp.int32)
    srcp = jnp.concatenate([src, psrc]).reshape(-1, 128)
    offs = jnp.arange(n_chunks, dtype=jnp.int32)[:, None, None]
    srcp = srcp[None] * n_chunks + offs
    dstp = jnp.concatenate([dst, pdst]).reshape(-1, 128)
    return srcp, dstp


def kernel(student_x, concept_x, lecture_x, und_src, und_dst, tea_src,
           tea_dst, W_fs, b_fs, W_fc, b_fc, W_fl, b_fl,
           Wu_self, Wu_neigh, bu, Wt_self, Wt_neigh, bt):
    NS, NC, NL = student_x.shape[0], concept_x.shape[0], lecture_x.shape[0]

    s0, s_out = _proj(student_x, W_fs, b_fs, with_relu=True)
    c0 = _proj(concept_x, W_fc, b_fc, with_relu=False)[0]
    l0 = _proj(lecture_x, W_fl, b_fl, with_relu=False)[0]

    und_srcp, und_dstp = _pad_edges(und_src, und_dst, NS, NC, n_chunks=8)
    tea_srcp, tea_dstp = _pad_edges(tea_src, tea_dst, NC, NL, n_chunks=2)

    U0c, U1c, degc_raw = _segsum(
        [s0.reshape(-1, 16), s_out.reshape(-1, 16)],
        und_srcp, und_dstp, NS, NC, w=16, n_chunks=8, with_deg=True)
    degc = degc_raw.reshape(NC, 1)

    T0c, degl_raw = _segsum([c0.reshape(-1, 64)],
                            tea_srcp, tea_dstp, NC, NL, w=64, n_chunks=2,
                            with_deg=True)
    degl = degl_raw.reshape(NL, 1)

    c1 = _combine(c0, U0c, degc, Wu_self[0], Wu_neigh[0], bu[0], relu=True)
    l1 = _combine(l0, T0c, degl, Wt_self[0], Wt_neigh[0], bt[0], relu=True)

    T1c, _ = _segsum([c1.reshape(-1, 64)], tea_srcp, tea_dstp, NC, NL,
                     w=64, n_chunks=2, with_deg=False)
    c2 = _combine(c1, U1c, degc, Wu_self[1], Wu_neigh[1], bu[1], relu=True)
    l2 = _combine(l1, T1c, degl, Wt_self[1], Wt_neigh[1], bt[1], relu=True)

    T2c, _ = _segsum([c2.reshape(-1, 64)], tea_srcp, tea_dstp, NC, NL,
                     w=64, n_chunks=2, with_deg=False)
    c3 = _combine(c2, U1c, degc, Wu_self[2], Wu_neigh[2], bu[2], relu=False)
    l3 = _combine(l2, T2c, degl, Wt_self[2], Wt_neigh[2], bt[2], relu=False)

    return (s_out, c3, l3)


# speedup vs baseline: 8.0768x; 1.3443x over previous
import functools

import jax
import jax.numpy as jnp
from jax import lax
from jax.experimental import pallas as pl
from jax.experimental.pallas import tpu as pltpu
from jax.experimental.pallas import tpu_sc as plsc

D = 128
_BN = 2000


def _rup(x, m):
    return (x + m - 1) // m * m


def _proj(x, W, b, with_relu):
    n = x.shape[0]

    def body(x_ref, w_ref, b_ref, *out_refs):
        y = jnp.dot(x_ref[...], w_ref[...], preferred_element_type=jnp.float32)
        y = y + b_ref[...]
        out_refs[0][...] = y
        if with_relu:
            out_refs[1][...] = jnp.maximum(y, 0.0)

    n_out = 2 if with_relu else 1
    return pl.pallas_call(
        body,
        grid=(n // _BN,),
        in_specs=[
            pl.BlockSpec((_BN, D), lambda i: (i, 0)),
            pl.BlockSpec((D, D), lambda i: (0, 0)),
            pl.BlockSpec((1, D), lambda i: (0, 0)),
        ],
        out_specs=[pl.BlockSpec((_BN, D), lambda i: (i, 0))] * n_out,
        out_shape=[jax.ShapeDtypeStruct((n, D), jnp.float32)] * n_out,
    )(x, W, b.reshape(1, D))


def _combine(x, ssum, deg2, W1, W2, b, relu):
    n = x.shape[0]

    def body(x_ref, s_ref, d_ref, w1_ref, w2_ref, b_ref, o_ref):
        rdeg = 1.0 / jnp.maximum(d_ref[...], 1.0)
        m = s_ref[...] * rdeg
        y = jnp.dot(x_ref[...], w1_ref[...], preferred_element_type=jnp.float32)
        y = y + jnp.dot(m, w2_ref[...], preferred_element_type=jnp.float32)
        y = y + b_ref[...]
        if relu:
            y = jnp.maximum(y, 0.0)
        o_ref[...] = y

    return pl.pallas_call(
        body,
        grid=(n // _BN,),
        in_specs=[
            pl.BlockSpec((_BN, D), lambda i: (i, 0)),
            pl.BlockSpec((_BN, D), lambda i: (i, 0)),
            pl.BlockSpec((_BN, 1), lambda i: (i, 0)),
            pl.BlockSpec((D, D), lambda i: (0, 0)),
            pl.BlockSpec((D, D), lambda i: (0, 0)),
            pl.BlockSpec((1, D), lambda i: (0, 0)),
        ],
        out_specs=pl.BlockSpec((_BN, D), lambda i: (i, 0)),
        out_shape=jax.ShapeDtypeStruct((n, D), jnp.float32),
    )(x, ssum, deg2, W1, W2, b.reshape(1, D))


_NSC = 2
_NTILE = 16
_K = 8
_ZR = 64


def _segsum(tables, srcp, dstp, n_src, n_dst, w, n_chunks, with_deg):
    n_tables = len(tables)
    e_rows = srcp.shape[1]
    assert e_rows % (_NTILE * _K) == 0
    tile_rows = e_rows // _NTILE
    n_groups = tile_rows // _K
    n_passes = n_chunks // _NSC
    nd_out = _rup(n_dst, _NTILE)
    wo_rows = nd_out // _NTILE
    dpt = _rup(-(-n_dst // _NTILE), 16)
    deg_n = _NTILE * dpt
    nd_acc = _rup(max(deg_n, nd_out, n_dst + 1), _NTILE * _ZR)
    zpt = nd_acc // _NTILE
    assert zpt % _ZR == 0

    mesh = plsc.VectorSubcoreMesh(core_axis_name="c", subcore_axis_name="s",
                                  num_cores=_NSC, num_subcores=_NTILE)

    out_type = [jax.ShapeDtypeStruct((nd_out, n_chunks * w), jnp.float32)
                for _ in range(n_tables)]
    out_type.append(jax.ShapeDtypeStruct((deg_n,), jnp.float32))

    scratch = []
    scratch += [pltpu.VMEM_SHARED((nd_acc, w), jnp.float32)
                for _ in range(n_tables)]
    scratch.append(pltpu.VMEM_SHARED((nd_acc,), jnp.float32))
    scratch.append(pltpu.VMEM((_K, 128), jnp.int32))
    scratch.append(pltpu.VMEM((_K, 128), jnp.int32))
    scratch.append(pltpu.VMEM((_K, 128, w), jnp.float32))
    scratch.append(pltpu.VMEM((_ZR, w), jnp.float32))
    scratch.append(pltpu.VMEM((_ZR,), jnp.float32))
    scratch.append(pltpu.VMEM((128,), jnp.float32))
    scratch.append(pltpu.SemaphoreType.DMA)

    @functools.partial(
        pl.kernel, mesh=mesh, out_type=out_type, scratch_types=scratch,
        compiler_params=pltpu.CompilerParams(use_tc_tiling_on_sc=False))
    def k(*refs):
        i = 0
        tbl = refs[i:i + n_tables]; i += n_tables
        src_hbm = refs[i]; i += 1
        dst_hbm = refs[i]; i += 1
        out = refs[i:i + n_tables]; i += n_tables
        deg_hbm = refs[i]; i += 1
        acc = refs[i:i + n_tables]; i += n_tables
        deg_acc = refs[i]; i += 1
        src_buf = refs[i]; i += 1
        dst_buf = refs[i]; i += 1
        msg = refs[i]; i += 1
        zbuf = refs[i]; i += 1
        zdeg = refs[i]; i += 1
        ones = refs[i]; i += 1
        sem = refs[i]; i += 1

        c = lax.axis_index("c")
        s = lax.axis_index("s")

        z16 = jnp.zeros((16,), jnp.float32)

        def zrow(r, _):
            for q in range(w // 16):
                zbuf[r, pl.ds(q * 16, 16)] = z16
            return 0

        lax.fori_loop(0, _ZR, zrow, 0)
        for q in range(_ZR // 16):
            zdeg[pl.ds(q * 16, 16)] = z16
        for q in range(128 // 16):
            ones[pl.ds(q * 16, 16)] = jnp.ones((16,), jnp.float32)

        zb = s * zpt

        def zero_accs():
            for t in range(n_tables):
                def zacc(g, _, _t=t):
                    pltpu.sync_copy(zbuf,
                                    acc[_t].at[pl.ds(zb + g * _ZR, _ZR), :])
                    return 0
                lax.fori_loop(0, zpt // _ZR, zacc, 0)

        zero_accs()
        if with_deg:
            @pl.when(c == 0)
            def _():
                def zd(g, _):
                    pltpu.sync_copy(zdeg, deg_acc.at[pl.ds(zb + g * _ZR, _ZR)])
                    return 0
                lax.fori_loop(0, zpt // _ZR, zd, 0)

        plsc.subcore_barrier()

        row0 = s * tile_rows
        wb = s * wo_rows
        for p in range(n_passes):
            j = c * n_passes + p

            def group(g, _, _p=p, _j=j):
                rb = row0 + g * _K
                c1 = pltpu.async_copy(src_hbm.at[_j, pl.ds(rb, _K)],
                                      src_buf, sem)
                c2 = pltpu.async_copy(dst_hbm.at[pl.ds(rb, _K)],
                                      dst_buf, sem)
                c1.wait()
                c2.wait()
                for t in range(n_tables):
                    cps = [pltpu.async_copy(
                        tbl[t].at[src_buf.at[r]], msg.at[r], sem)
                        for r in range(_K)]
                    for cp in cps:
                        cp.wait()
                    cps = [pltpu.async_copy(
                        msg.at[r], acc[t].at[dst_buf.at[r]], sem, add=True)
                        for r in range(_K)]
                    for cp in cps:
                        cp.wait()
                if with_deg and _p == 0:
                    @pl.when(c == 0)
                    def _():
                        dc = [pltpu.async_copy(
                            ones, deg_acc.at[dst_buf.at[r]], sem, add=True)
                            for r in range(_K)]
                        for cp in dc:
                            cp.wait()
                return 0

            lax.fori_loop(0, n_groups, group, 0)
            plsc.subcore_barrier()
            for t in range(n_tables):
                pltpu.sync_copy(acc[t].at[pl.ds(wb, wo_rows), :],
                                out[t].at[pl.ds(wb, wo_rows),
                                          pl.ds(j * w, w)])
            if p != n_passes - 1:
                plsc.subcore_barrier()
                zero_accs()
                plsc.subcore_barrier()

        if with_deg:
            @pl.when(c == 0)
            def _():
                db = s * dpt
                pltpu.sync_copy(deg_acc.at[pl.ds(db, dpt)],
                                deg_hbm.at[pl.ds(db, dpt)])

    res = k(*tables, srcp, dstp)
    return tuple(o[:n_dst] for o in res[:-1]) + (res[-1][:n_dst],)


def _pad_edges(src, dst, n_src, n_dst, n_chunks):
    e = src.shape[0]
    e_pad = _rup(e, _NTILE * _K * 128)
    pad = e_pad - e
    psrc = jnp.arange(pad, dtype=jnp.int32) % n_src
    pdst = jnp.full((pad,), n_dst, jnp.int32)
    srcp = jnp.concatenate([src, psrc]).reshape(-1, 128)
    offs = jnp.arange(n_chunks, dtype=jnp.int32)[:, None, None]
    srcp = srcp[None] * n_chunks + offs
    dstp = jnp.concatenate([dst, pdst]).reshape(-1, 128)
    return srcp, dstp


def kernel(student_x, concept_x, lecture_x, und_src, und_dst, tea_src,
           tea_dst, W_fs, b_fs, W_fc, b_fc, W_fl, b_fl,
           Wu_self, Wu_neigh, bu, Wt_self, Wt_neigh, bt):
    NS, NC, NL = student_x.shape[0], concept_x.shape[0], lecture_x.shape[0]

    s0, s_out = _proj(student_x, W_fs, b_fs, with_relu=True)
    c0 = _proj(concept_x, W_fc, b_fc, with_relu=False)[0]
    l0 = _proj(lecture_x, W_fl, b_fl, with_relu=False)[0]

    und_srcp, und_dstp = _pad_edges(und_src, und_dst, NS, NC, n_chunks=8)
    tea_srcp, tea_dstp = _pad_edges(tea_src, tea_dst, NC, NL, n_chunks=2)

    U0c, U1c, degc_raw = _segsum(
        [s0.reshape(-1, 16), s_out.reshape(-1, 16)],
        und_srcp, und_dstp, NS, NC, w=16, n_chunks=8, with_deg=True)
    degc = degc_raw.reshape(NC, 1)

    T0c, degl_raw = _segsum([c0.reshape(-1, 64)],
                            tea_srcp, tea_dstp, NC, NL, w=64, n_chunks=2,
                            with_deg=True)
    degl = degl_raw.reshape(NL, 1)

    c1 = _combine(c0, U0c, degc, Wu_self[0], Wu_neigh[0], bu[0], relu=True)
    l1 = _combine(l0, T0c, degl, Wt_self[0], Wt_neigh[0], bt[0], relu=True)

    T1c, _ = _segsum([c1.reshape(-1, 64)], tea_srcp, tea_dstp, NC, NL,
                     w=64, n_chunks=2, with_deg=False)
    c2 = _combine(c1, U1c, degc, Wu_self[1], Wu_neigh[1], bu[1], relu=True)
    l2 = _combine(l1, T1c, degl, Wt_self[1], Wt_neigh[1], bt[1], relu=True)

    T2c, _ = _segsum([c2.reshape(-1, 64)], tea_srcp, tea_dstp, NC, NL,
                     w=64, n_chunks=2, with_deg=False)
    c3 = _combine(c2, U1c, degc, Wu_self[2], Wu_neigh[2], bu[2], relu=False)
    l3 = _combine(l2, T2c, degl, Wt_self[2], Wt_neigh[2], bt[2], relu=False)

    return (s_out, c3, l3)


# speedup vs baseline: 8.5794x; 1.0622x over previous
import functools

import jax
import jax.numpy as jnp
from jax import lax
from jax.experimental import pallas as pl
from jax.experimental.pallas import tpu as pltpu
from jax.experimental.pallas import tpu_sc as plsc

D = 128
_BN = 2000


def _rup(x, m):
    return (x + m - 1) // m * m


def _proj(x, W, b, with_relu):
    n = x.shape[0]

    def body(x_ref, w_ref, b_ref, *out_refs):
        y = jnp.dot(x_ref[...], w_ref[...], preferred_element_type=jnp.float32)
        y = y + b_ref[...]
        out_refs[0][...] = y
        if with_relu:
            out_refs[1][...] = jnp.maximum(y, 0.0)

    n_out = 2 if with_relu else 1
    return pl.pallas_call(
        body,
        grid=(n // _BN,),
        in_specs=[
            pl.BlockSpec((_BN, D), lambda i: (i, 0)),
            pl.BlockSpec((D, D), lambda i: (0, 0)),
            pl.BlockSpec((1, D), lambda i: (0, 0)),
        ],
        out_specs=[pl.BlockSpec((_BN, D), lambda i: (i, 0))] * n_out,
        out_shape=[jax.ShapeDtypeStruct((n, D), jnp.float32)] * n_out,
    )(x, W, b.reshape(1, D))


def _combine(x, ssum, deg2, W1, W2, b, relu):
    n = x.shape[0]

    def body(x_ref, s_ref, d_ref, w1_ref, w2_ref, b_ref, o_ref):
        rdeg = 1.0 / jnp.maximum(d_ref[...], 1.0)
        m = s_ref[...] * rdeg
        y = jnp.dot(x_ref[...], w1_ref[...], preferred_element_type=jnp.float32)
        y = y + jnp.dot(m, w2_ref[...], preferred_element_type=jnp.float32)
        y = y + b_ref[...]
        if relu:
            y = jnp.maximum(y, 0.0)
        o_ref[...] = y

    return pl.pallas_call(
        body,
        grid=(n // _BN,),
        in_specs=[
            pl.BlockSpec((_BN, D), lambda i: (i, 0)),
            pl.BlockSpec((_BN, D), lambda i: (i, 0)),
            pl.BlockSpec((_BN, 1), lambda i: (i, 0)),
            pl.BlockSpec((D, D), lambda i: (0, 0)),
            pl.BlockSpec((D, D), lambda i: (0, 0)),
            pl.BlockSpec((1, D), lambda i: (0, 0)),
        ],
        out_specs=pl.BlockSpec((_BN, D), lambda i: (i, 0)),
        out_shape=jax.ShapeDtypeStruct((n, D), jnp.float32),
    )(x, ssum, deg2, W1, W2, b.reshape(1, D))


_NSC = 2
_NTILE = 16
_K = 8
_ZR = 64


def _segsum(tables, srcp, dstp, n_src, n_dst, w, n_chunks, with_deg):
    n_tables = len(tables)
    e_rows = srcp.shape[1]
    assert e_rows % (_NTILE * _K) == 0
    tile_rows = e_rows // _NTILE
    n_groups = tile_rows // _K
    n_passes = n_chunks // _NSC
    nd_out = _rup(n_dst, _NTILE)
    wo_rows = nd_out // _NTILE
    dpt = _rup(-(-n_dst // _NTILE), 16)
    deg_n = _NTILE * dpt
    nd_acc = _rup(max(deg_n, nd_out, n_dst + 1), _NTILE * _ZR)
    zpt = nd_acc // _NTILE
    assert zpt % _ZR == 0

    mesh = plsc.VectorSubcoreMesh(core_axis_name="c", subcore_axis_name="s",
                                  num_cores=_NSC, num_subcores=_NTILE)

    out_type = [jax.ShapeDtypeStruct((nd_out, n_chunks * w), jnp.float32)
                for _ in range(n_tables)]
    out_type.append(jax.ShapeDtypeStruct((deg_n,), jnp.float32))

    scratch = []
    scratch += [pltpu.VMEM_SHARED((nd_acc, w), jnp.float32)
                for _ in range(n_tables)]
    scratch.append(pltpu.VMEM_SHARED((nd_acc,), jnp.float32))
    scratch.append(pltpu.VMEM((2, _K, 128), jnp.int32))
    scratch.append(pltpu.VMEM((2, _K, 128), jnp.int32))
    scratch.append(pltpu.VMEM((_K, 128, w), jnp.float32))
    scratch.append(pltpu.SemaphoreType.DMA)
    scratch.append(pltpu.VMEM((_ZR, w), jnp.float32))
    scratch.append(pltpu.VMEM((_ZR,), jnp.float32))
    scratch.append(pltpu.VMEM((128,), jnp.float32))
    scratch.append(pltpu.SemaphoreType.DMA)

    @functools.partial(
        pl.kernel, mesh=mesh, out_type=out_type, scratch_types=scratch,
        compiler_params=pltpu.CompilerParams(use_tc_tiling_on_sc=False))
    def k(*refs):
        i = 0
        tbl = refs[i:i + n_tables]; i += n_tables
        src_hbm = refs[i]; i += 1
        dst_hbm = refs[i]; i += 1
        out = refs[i:i + n_tables]; i += n_tables
        deg_hbm = refs[i]; i += 1
        acc = refs[i:i + n_tables]; i += n_tables
        deg_acc = refs[i]; i += 1
        src_buf = refs[i]; i += 1
        dst_buf = refs[i]; i += 1
        msg = refs[i]; i += 1
        isem = refs[i]; i += 1
        zbuf = refs[i]; i += 1
        zdeg = refs[i]; i += 1
        ones = refs[i]; i += 1
        sem = refs[i]; i += 1

        c = lax.axis_index("c")
        s = lax.axis_index("s")

        z16 = jnp.zeros((16,), jnp.float32)

        def zrow(r, _):
            for q in range(w // 16):
                zbuf[r, pl.ds(q * 16, 16)] = z16
            return 0

        lax.fori_loop(0, _ZR, zrow, 0)
        for q in range(_ZR // 16):
            zdeg[pl.ds(q * 16, 16)] = z16
        for q in range(128 // 16):
            ones[pl.ds(q * 16, 16)] = jnp.ones((16,), jnp.float32)

        zb = s * zpt

        def zero_accs():
            for t in range(n_tables):
                def zacc(g, _, _t=t):
                    pltpu.sync_copy(zbuf,
                                    acc[_t].at[pl.ds(zb + g * _ZR, _ZR), :])
                    return 0
                lax.fori_loop(0, zpt // _ZR, zacc, 0)

        zero_accs()
        if with_deg:
            @pl.when(c == 0)
            def _():
                def zd(g, _):
                    pltpu.sync_copy(zdeg, deg_acc.at[pl.ds(zb + g * _ZR, _ZR)])
                    return 0
                lax.fori_loop(0, zpt // _ZR, zd, 0)

        plsc.subcore_barrier()

        assert n_groups % 2 == 1
        row0 = s * tile_rows
        wb = s * wo_rows
        for p in range(n_passes):
            j = c * n_passes + p

            def fire_idx(g, pi, _j=j):
                rb = row0 + g * _K
                pltpu.async_copy(src_hbm.at[_j, pl.ds(rb, _K)],
                                 src_buf.at[pi], isem)
                pltpu.async_copy(dst_hbm.at[pl.ds(rb, _K)],
                                 dst_buf.at[pi], isem)

            def wait_idx(pi):
                pltpu.make_async_copy(src_hbm.at[0, pl.ds(0, _K)],
                                      src_buf.at[pi], isem).wait()
                pltpu.make_async_copy(dst_hbm.at[pl.ds(0, _K)],
                                      dst_buf.at[pi], isem).wait()

            def process(pi, _p=p):
                for t in range(n_tables):
                    cps = [pltpu.async_copy(
                        tbl[t].at[src_buf.at[pi, r]], msg.at[r], sem)
                        for r in range(_K)]
                    for cp in cps:
                        cp.wait()
                    cps = [pltpu.async_copy(
                        msg.at[r], acc[t].at[dst_buf.at[pi, r]], sem,
                        add=True)
                        for r in range(_K)]
                    for cp in cps:
                        cp.wait()
                if with_deg and _p == 0:
                    @pl.when(c == 0)
                    def _():
                        dc = [pltpu.async_copy(
                            ones, deg_acc.at[dst_buf.at[pi, r]], sem,
                            add=True)
                            for r in range(_K)]
                        for cp in dc:
                            cp.wait()

            fire_idx(0, 0)

            def pair(h, _):
                wait_idx(0)
                fire_idx(2 * h + 1, 1)
                process(0)
                wait_idx(1)
                fire_idx(2 * h + 2, 0)
                process(1)
                return 0

            lax.fori_loop(0, (n_groups - 1) // 2, pair, 0)
            wait_idx(0)
            process(0)
            plsc.subcore_barrier()
            for t in range(n_tables):
                pltpu.sync_copy(acc[t].at[pl.ds(wb, wo_rows), :],
                                out[t].at[pl.ds(wb, wo_rows),
                                          pl.ds(j * w, w)])
            if p != n_passes - 1:
                plsc.subcore_barrier()
                zero_accs()
                plsc.subcore_barrier()

        if with_deg:
            @pl.when(c == 0)
            def _():
                db = s * dpt
                pltpu.sync_copy(deg_acc.at[pl.ds(db, dpt)],
                                deg_hbm.at[pl.ds(db, dpt)])

    res = k(*tables, srcp, dstp)
    return tuple(o[:n_dst] for o in res[:-1]) + (res[-1][:n_dst],)


def _pad_edges(src, dst, n_src, n_dst, n_chunks):
    e = src.shape[0]
    e_pad = _rup(e, _NTILE * _K * 128)
    pad = e_pad - e
    psrc = jnp.arange(pad, dtype=jnp.int32) % n_src
    pdst = jnp.full((pad,), n_dst, jnp.int32)
    srcp = jnp.concatenate([src, psrc]).reshape(-1, 128)
    offs = jnp.arange(n_chunks, dtype=jnp.int32)[:, None, None]
    srcp = srcp[None] * n_chunks + offs
    dstp = jnp.concatenate([dst, pdst]).reshape(-1, 128)
    return srcp, dstp


def kernel(student_x, concept_x, lecture_x, und_src, und_dst, tea_src,
           tea_dst, W_fs, b_fs, W_fc, b_fc, W_fl, b_fl,
           Wu_self, Wu_neigh, bu, Wt_self, Wt_neigh, bt):
    NS, NC, NL = student_x.shape[0], concept_x.shape[0], lecture_x.shape[0]

    s0, s_out = _proj(student_x, W_fs, b_fs, with_relu=True)
    c0 = _proj(concept_x, W_fc, b_fc, with_relu=False)[0]
    l0 = _proj(lecture_x, W_fl, b_fl, with_relu=False)[0]

    und_srcp, und_dstp = _pad_edges(und_src, und_dst, NS, NC, n_chunks=8)
    tea_srcp, tea_dstp = _pad_edges(tea_src, tea_dst, NC, NL, n_chunks=2)

    U0c, U1c, degc_raw = _segsum(
        [s0.reshape(-1, 16), s_out.reshape(-1, 16)],
        und_srcp, und_dstp, NS, NC, w=16, n_chunks=8, with_deg=True)
    degc = degc_raw.reshape(NC, 1)

    T0c, degl_raw = _segsum([c0.reshape(-1, 64)],
                            tea_srcp, tea_dstp, NC, NL, w=64, n_chunks=2,
                            with_deg=True)
    degl = degl_raw.reshape(NL, 1)

    c1 = _combine(c0, U0c, degc, Wu_self[0], Wu_neigh[0], bu[0], relu=True)
    l1 = _combine(l0, T0c, degl, Wt_self[0], Wt_neigh[0], bt[0], relu=True)

    T1c, _ = _segsum([c1.reshape(-1, 64)], tea_srcp, tea_dstp, NC, NL,
                     w=64, n_chunks=2, with_deg=False)
    c2 = _combine(c1, U1c, degc, Wu_self[1], Wu_neigh[1], bu[1], relu=True)
    l2 = _combine(l1, T1c, degl, Wt_self[1], Wt_neigh[1], bt[1], relu=True)

    T2c, _ = _segsum([c2.reshape(-1, 64)], tea_srcp, tea_dstp, NC, NL,
                     w=64, n_chunks=2, with_deg=False)
    c3 = _combine(c2, U1c, degc, Wu_self[2], Wu_neigh[2], bu[2], relu=False)
    l3 = _combine(l2, T2c, degl, Wt_self[2], Wt_neigh[2], bt[2], relu=False)

    return (s_out, c3, l3)


# speedup vs baseline: 8.6686x; 1.0104x over previous
import functools

import jax
import jax.numpy as jnp
from jax import lax
from jax.experimental import pallas as pl
from jax.experimental.pallas import tpu as pltpu
from jax.experimental.pallas import tpu_sc as plsc

D = 128
_BN = 2000


def _rup(x, m):
    return (x + m - 1) // m * m


def _proj(x, W, b, with_relu):
    n = x.shape[0]

    def body(x_ref, w_ref, b_ref, *out_refs):
        y = jnp.dot(x_ref[...], w_ref[...], preferred_element_type=jnp.float32)
        y = y + b_ref[...]
        out_refs[0][...] = y
        if with_relu:
            out_refs[1][...] = jnp.maximum(y, 0.0)

    n_out = 2 if with_relu else 1
    return pl.pallas_call(
        body,
        grid=(n // _BN,),
        in_specs=[
            pl.BlockSpec((_BN, D), lambda i: (i, 0)),
            pl.BlockSpec((D, D), lambda i: (0, 0)),
            pl.BlockSpec((1, D), lambda i: (0, 0)),
        ],
        out_specs=[pl.BlockSpec((_BN, D), lambda i: (i, 0))] * n_out,
        out_shape=[jax.ShapeDtypeStruct((n, D), jnp.float32)] * n_out,
    )(x, W, b.reshape(1, D))


def _combine(x, ssum, deg2, W1, W2, b, relu):
    n = x.shape[0]

    def body(x_ref, s_ref, d_ref, w1_ref, w2_ref, b_ref, o_ref):
        rdeg = 1.0 / jnp.maximum(d_ref[...], 1.0)
        m = s_ref[...] * rdeg
        y = jnp.dot(x_ref[...], w1_ref[...], preferred_element_type=jnp.float32)
        y = y + jnp.dot(m, w2_ref[...], preferred_element_type=jnp.float32)
        y = y + b_ref[...]
        if relu:
            y = jnp.maximum(y, 0.0)
        o_ref[...] = y

    return pl.pallas_call(
        body,
        grid=(n // _BN,),
        in_specs=[
            pl.BlockSpec((_BN, D), lambda i: (i, 0)),
            pl.BlockSpec((_BN, D), lambda i: (i, 0)),
            pl.BlockSpec((_BN, 1), lambda i: (i, 0)),
            pl.BlockSpec((D, D), lambda i: (0, 0)),
            pl.BlockSpec((D, D), lambda i: (0, 0)),
            pl.BlockSpec((1, D), lambda i: (0, 0)),
        ],
        out_specs=pl.BlockSpec((_BN, D), lambda i: (i, 0)),
        out_shape=jax.ShapeDtypeStruct((n, D), jnp.float32),
    )(x, ssum, deg2, W1, W2, b.reshape(1, D))


_NSC = 2
_NTILE = 16
_K = 8
_ZR = 64


def _segsum(tables, srcp, dstp, n_src, n_dst, w, n_chunks, with_deg):
    n_tables = len(tables)
    e_rows = srcp.shape[1]
    assert e_rows % (_NTILE * _K) == 0
    tile_rows = e_rows // _NTILE
    n_groups = tile_rows // _K
    n_passes = n_chunks // _NSC
    nd_out = _rup(n_dst, _NTILE)
    wo_rows = nd_out // _NTILE
    dpt = _rup(-(-n_dst // _NTILE), 16)
    deg_n = _NTILE * dpt
    nd_acc = _rup(max(deg_n, nd_out, n_dst + 1), _NTILE * _ZR)
    zpt = nd_acc // _NTILE
    assert zpt % _ZR == 0

    mesh = plsc.VectorSubcoreMesh(core_axis_name="c", subcore_axis_name="s",
                                  num_cores=_NSC, num_subcores=_NTILE)

    out_type = [jax.ShapeDtypeStruct((nd_out, n_chunks * w), jnp.float32)
                for _ in range(n_tables)]
    out_type.append(jax.ShapeDtypeStruct((deg_n,), jnp.float32))

    scratch = []
    scratch += [pltpu.VMEM_SHARED((nd_acc, w), jnp.float32)
                for _ in range(n_tables)]
    scratch.append(pltpu.VMEM_SHARED((nd_acc,), jnp.float32))
    scratch.append(pltpu.VMEM((2, _K, 128), jnp.int32))
    scratch.append(pltpu.VMEM((2, _K, 128), jnp.int32))
    scratch.append(pltpu.VMEM((_K, 128, w), jnp.float32))
    scratch.append(pltpu.SemaphoreType.DMA)
    scratch.append(pltpu.VMEM((_ZR, w), jnp.float32))
    scratch.append(pltpu.VMEM((_ZR,), jnp.float32))
    scratch.append(pltpu.VMEM((128,), jnp.float32))
    scratch.append(pltpu.SemaphoreType.DMA)

    @functools.partial(
        pl.kernel, mesh=mesh, out_type=out_type, scratch_types=scratch,
        compiler_params=pltpu.CompilerParams(use_tc_tiling_on_sc=False))
    def k(*refs):
        i = 0
        tbl = refs[i:i + n_tables]; i += n_tables
        src_hbm = refs[i]; i += 1
        dst_hbm = refs[i]; i += 1
        out = refs[i:i + n_tables]; i += n_tables
        deg_hbm = refs[i]; i += 1
        acc = refs[i:i + n_tables]; i += n_tables
        deg_acc = refs[i]; i += 1
        src_buf = refs[i]; i += 1
        dst_buf = refs[i]; i += 1
        msg = refs[i]; i += 1
        isem = refs[i]; i += 1
        zbuf = refs[i]; i += 1
        zdeg = refs[i]; i += 1
        ones = refs[i]; i += 1
        sem = refs[i]; i += 1

        c = lax.axis_index("c")
        s = lax.axis_index("s")

        z16 = jnp.zeros((16,), jnp.float32)

        def zrow(r, _):
            for q in range(w // 16):
                zbuf[r, pl.ds(q * 16, 16)] = z16
            return 0

        lax.fori_loop(0, _ZR, zrow, 0)
        for q in range(_ZR // 16):
            zdeg[pl.ds(q * 16, 16)] = z16
        for q in range(128 // 16):
            ones[pl.ds(q * 16, 16)] = jnp.ones((16,), jnp.float32)

        zb = s * zpt

        def zero_accs():
            cps = []
            for t in range(n_tables):
                for g in range(zpt // _ZR):
                    cps.append(pltpu.async_copy(
                        zbuf, acc[t].at[pl.ds(zb + g * _ZR, _ZR), :], sem))
            for cp in cps:
                cp.wait()

        zero_accs()
        if with_deg:
            @pl.when(c == 0)
            def _():
                cps = [pltpu.async_copy(
                    zdeg, deg_acc.at[pl.ds(zb + g * _ZR, _ZR)], sem)
                    for g in range(zpt // _ZR)]
                for cp in cps:
                    cp.wait()

        plsc.subcore_barrier()

        assert n_groups % 2 == 1
        row0 = s * tile_rows
        wb = s * wo_rows
        for p in range(n_passes):
            j = c * n_passes + p

            def fire_idx(g, pi, _j=j):
                rb = row0 + g * _K
                pltpu.async_copy(src_hbm.at[_j, pl.ds(rb, _K)],
                                 src_buf.at[pi], isem)
                pltpu.async_copy(dst_hbm.at[pl.ds(rb, _K)],
                                 dst_buf.at[pi], isem)

            def wait_idx(pi):
                pltpu.make_async_copy(src_hbm.at[0, pl.ds(0, _K)],
                                      src_buf.at[pi], isem).wait()
                pltpu.make_async_copy(dst_hbm.at[pl.ds(0, _K)],
                                      dst_buf.at[pi], isem).wait()

            def process(pi, _p=p):
                for t in range(n_tables):
                    cps = [pltpu.async_copy(
                        tbl[t].at[src_buf.at[pi, r]], msg.at[r], sem)
                        for r in range(_K)]
                    for cp in cps:
                        cp.wait()
                    cps = [pltpu.async_copy(
                        msg.at[r], acc[t].at[dst_buf.at[pi, r]], sem,
                        add=True)
                        for r in range(_K)]
                    for cp in cps:
                        cp.wait()
                if with_deg and _p == 0:
                    @pl.when(c == 0)
                    def _():
                        dc = [pltpu.async_copy(
                            ones, deg_acc.at[dst_buf.at[pi, r]], sem,
                            add=True)
                            for r in range(_K)]
                        for cp in dc:
                            cp.wait()

            fire_idx(0, 0)

            def pair(h, _):
                wait_idx(0)
                fire_idx(2 * h + 1, 1)
                process(0)
                wait_idx(1)
                fire_idx(2 * h + 2, 0)
                process(1)
                return 0

            lax.fori_loop(0, (n_groups - 1) // 2, pair, 0)
            wait_idx(0)
            process(0)
            plsc.subcore_barrier()
            for t in range(n_tables):
                pltpu.sync_copy(acc[t].at[pl.ds(wb, wo_rows), :],
                                out[t].at[pl.ds(wb, wo_rows),
                                          pl.ds(j * w, w)])
            if p != n_passes - 1:
                plsc.subcore_barrier()
                zero_accs()
                plsc.subcore_barrier()

        if with_deg:
            @pl.when(c == 0)
            def _():
                db = s * dpt
                pltpu.sync_copy(deg_acc.at[pl.ds(db, dpt)],
                                deg_hbm.at[pl.ds(db, dpt)])

    res = k(*tables, srcp, dstp)
    return tuple(o[:n_dst] for o in res[:-1]) + (res[-1][:n_dst],)


def _pad_edges(src, dst, n_src, n_dst, n_chunks):
    e = src.shape[0]
    e_pad = _rup(e, _NTILE * _K * 128)
    pad = e_pad - e
    psrc = jnp.arange(pad, dtype=jnp.int32) % n_src
    pdst = jnp.full((pad,), n_dst, jnp.int32)
    srcp = jnp.concatenate([src, psrc]).reshape(-1, 128)
    offs = jnp.arange(n_chunks, dtype=jnp.int32)[:, None, None]
    srcp = srcp[None] * n_chunks + offs
    dstp = jnp.concatenate([dst, pdst]).reshape(-1, 128)
    return srcp, dstp


def kernel(student_x, concept_x, lecture_x, und_src, und_dst, tea_src,
           tea_dst, W_fs, b_fs, W_fc, b_fc, W_fl, b_fl,
           Wu_self, Wu_neigh, bu, Wt_self, Wt_neigh, bt):
    NS, NC, NL = student_x.shape[0], concept_x.shape[0], lecture_x.shape[0]

    s0, s_out = _proj(student_x, W_fs, b_fs, with_relu=True)
    c0 = _proj(concept_x, W_fc, b_fc, with_relu=False)[0]
    l0 = _proj(lecture_x, W_fl, b_fl, with_relu=False)[0]

    und_srcp, und_dstp = _pad_edges(und_src, und_dst, NS, NC, n_chunks=8)
    tea_srcp, tea_dstp = _pad_edges(tea_src, tea_dst, NC, NL, n_chunks=2)

    U0c, U1c, degc_raw = _segsum(
        [s0.reshape(-1, 16), s_out.reshape(-1, 16)],
        und_srcp, und_dstp, NS, NC, w=16, n_chunks=8, with_deg=True)
    degc = degc_raw.reshape(NC, 1)

    T0c, degl_raw = _segsum([c0.reshape(-1, 64)],
                            tea_srcp, tea_dstp, NC, NL, w=64, n_chunks=2,
                            with_deg=True)
    degl = degl_raw.reshape(NL, 1)

    c1 = _combine(c0, U0c, degc, Wu_self[0], Wu_neigh[0], bu[0], relu=True)
    l1 = _combine(l0, T0c, degl, Wt_self[0], Wt_neigh[0], bt[0], relu=True)

    T1c, _ = _segsum([c1.reshape(-1, 64)], tea_srcp, tea_dstp, NC, NL,
                     w=64, n_chunks=2, with_deg=False)
    c2 = _combine(c1, U1c, degc, Wu_self[1], Wu_neigh[1], bu[1], relu=True)
    l2 = _combine(l1, T1c, degl, Wt_self[1], Wt_neigh[1], bt[1], relu=True)

    T2c, _ = _segsum([c2.reshape(-1, 64)], tea_srcp, tea_dstp, NC, NL,
                     w=64, n_chunks=2, with_deg=False)
    c3 = _combine(c2, U1c, degc, Wu_self[2], Wu_neigh[2], bu[2], relu=False)
    l3 = _combine(l2, T2c, degl, Wt_self[2], Wt_neigh[2], bt[2], relu=False)

    return (s_out, c3, l3)
